# Initial kernel scaffold; baseline (speedup 1.0000x reference)
import jax
import jax.numpy as jnp
from jax.experimental import pallas as pl


def kernel(z, edge_index, lr_scores, params):
    raise NotImplementedError("write your pallas kernel here")


# speedup vs baseline: 5.3578x; 5.3578x over previous
import functools

import jax
import jax.numpy as jnp
from jax import lax
from jax.experimental import pallas as pl
from jax.experimental.pallas import tpu as pltpu
from jax.experimental.pallas import tpu_sc as plsc

N = 10000
E = 320000
D = 128
NF = 8
FE = 64

_NC = 2
_NS = 16
_NW = _NC * _NS
_EPW = E // _NW
_CH = 80
_NCHUNK = _EPW // _CH
_NPAD = 10240
_RPT = _NPAD // _NS
_MW = D + 16

_f32 = jnp.float32


_SQRT_HALF = 0.7071067811865476


def _gelu(x):
    return 0.5 * x * (1.0 + lax.erf(x * _SQRT_HALF))


def _tables_body(z_ref, wa_ref, ba_ref, wbc_ref, bbc_ref, a_ref, bc_ref):
    z = z_ref[...]
    a_ref[...] = jnp.dot(z, wa_ref[...], preferred_element_type=_f32) + ba_ref[...]
    bc_ref[...] = jnp.dot(z, wbc_ref[...], preferred_element_type=_f32) + bbc_ref[...]


def _make_tables(z, wa, ba, wbc, bbc):
    bn = 2000
    return pl.pallas_call(
        _tables_body,
        grid=(N // bn,),
        in_specs=[
            pl.BlockSpec((bn, D), lambda i: (i, 0)),
            pl.BlockSpec((D, D), lambda i: (0, 0)),
            pl.BlockSpec((1, D), lambda i: (0, 0)),
            pl.BlockSpec((D, 2 * D), lambda i: (0, 0)),
            pl.BlockSpec((1, 2 * D), lambda i: (0, 0)),
        ],
        out_specs=[
            pl.BlockSpec((bn, D), lambda i: (i, 0)),
            pl.BlockSpec((bn, 2 * D), lambda i: (i, 0)),
        ],
        out_shape=[
            jax.ShapeDtypeStruct((N, D), _f32),
            jax.ShapeDtypeStruct((N, 2 * D), _f32),
        ],
    )(z, wa, ba, wbc, bbc)


def _sc_gather_body(a_hbm, bc_hbm, src_hbm, dst_hbm, ga_hbm, gbc_hbm,
                    src_v, dst_v, buf_a, buf_bc, sem_a, sem_bc):
    c = lax.axis_index("c")
    s = lax.axis_index("s")
    wid = s * _NC + c
    base = wid * _EPW

    def chunk(i, carry):
        off = base + i * _CH
        pltpu.sync_copy(src_hbm.at[pl.ds(off, _CH)], src_v)
        pltpu.sync_copy(dst_hbm.at[pl.ds(off, _CH)], dst_v)
        cp_a = pltpu.async_copy(a_hbm.at[src_v], buf_a, sem_a)
        cp_bc = pltpu.async_copy(bc_hbm.at[dst_v], buf_bc, sem_bc)
        cp_a.wait()
        cp_bc.wait()
        pltpu.sync_copy(buf_a, ga_hbm.at[pl.ds(off, _CH)])
        pltpu.sync_copy(buf_bc, gbc_hbm.at[pl.ds(off, _CH)])
        return carry

    lax.fori_loop(0, _NCHUNK, chunk, 0)


def _sc_gather(a_tab, bc_tab, src, dst):
    mesh = plsc.VectorSubcoreMesh(
        core_axis_name="c", subcore_axis_name="s",
        num_cores=_NC, num_subcores=_NS)
    fn = pl.kernel(
        _sc_gather_body,
        out_type=[
            jax.ShapeDtypeStruct((E, D), _f32),
            jax.ShapeDtypeStruct((E, 2 * D), _f32),
        ],
        mesh=mesh,
        scratch_types=[
            pltpu.VMEM((_CH,), jnp.int32),
            pltpu.VMEM((_CH,), jnp.int32),
            pltpu.VMEM((_CH, D), _f32),
            pltpu.VMEM((_CH, 2 * D), _f32),
            pltpu.SemaphoreType.DMA,
            pltpu.SemaphoreType.DMA,
        ],
    )
    return fn(a_tab, bc_tab, src, dst)


def _edge_body(ga_ref, gbc_ref, lr_ref, w1c_ref, w2_ref, b2_ref,
               amw1b_ref, amw2_ref, amb2_ref, out_ref, ex_ref):
    ga = ga_ref[...]
    gbc = gbc_ref[...]
    lr = lr_ref[...]
    g = ga + gbc[:, :D]
    cd = gbc[:, D:]
    h = _gelu(g + lr * w1c_ref[...])
    le = jnp.dot(h, w2_ref[...], preferred_element_type=_f32) + b2_ref[...]
    t = _gelu(cd + jnp.dot(le, amw1b_ref[...], preferred_element_type=_f32))
    sc = jnp.sum(t * amw2_ref[...], axis=1, keepdims=True) + amb2_ref[...]
    ex = jnp.exp(sc)
    out_ref[...] = le * (ex * lr)
    ex_ref[...] = ex


def _edge_stage(ga, gbc, lr2, w1c, w2, b2, amw1b, amw2row, amb2):
    bb = 512
    return pl.pallas_call(
        _edge_body,
        grid=(E // bb,),
        in_specs=[
            pl.BlockSpec((bb, D), lambda i: (i, 0)),
            pl.BlockSpec((bb, 2 * D), lambda i: (i, 0)),
            pl.BlockSpec((bb, 1), lambda i: (i, 0)),
            pl.BlockSpec((1, D), lambda i: (0, 0)),
            pl.BlockSpec((D, D), lambda i: (0, 0)),
            pl.BlockSpec((1, D), lambda i: (0, 0)),
            pl.BlockSpec((D, D), lambda i: (0, 0)),
            pl.BlockSpec((1, D), lambda i: (0, 0)),
            pl.BlockSpec((1, 1), lambda i: (0, 0)),
        ],
        out_specs=[
            pl.BlockSpec((bb, D), lambda i: (i, 0)),
            pl.BlockSpec((bb, 1), lambda i: (i, 0)),
        ],
        out_shape=[
            jax.ShapeDtypeStruct((E, D), _f32),
            jax.ShapeDtypeStruct((E, 1), _f32),
        ],
    )(ga, gbc, lr2, w1c, w2, b2, amw1b, amw2row, amb2)


def _sc_scatter_body(msg_hbm, ex_hbm, dst_hbm, zero2_hbm, zero1_hbm,
                     outp_hbm, outd_hbm, idx_v, buf, ex_v, den, acc, sem):
    c = lax.axis_index("c")
    s = lax.axis_index("s")
    wid = s * _NC + c
    base = wid * _EPW
    r0 = s * _RPT
    pltpu.sync_copy(zero2_hbm.at[pl.ds(r0, _RPT)], acc.at[pl.ds(r0, _RPT)])
    pltpu.sync_copy(zero1_hbm, den)
    plsc.subcore_barrier()

    def chunk(i, carry):
        off = base + i * _CH
        pltpu.sync_copy(dst_hbm.at[pl.ds(off, _CH)], idx_v)
        pltpu.sync_copy(msg_hbm.at[pl.ds(off, _CH)], buf)
        pltpu.sync_copy(ex_hbm.at[pl.ds(off, _CH)], ex_v)
        pltpu.sync_copy(buf, acc.at[idx_v], add=True)
        for j in range(_CH // 16):
            sl = pl.ds(j * 16, 16)
            plsc.addupdate_scatter(den, [idx_v[sl]], ex_v[sl])
        return carry

    lax.fori_loop(0, _NCHUNK, chunk, 0)
    plsc.subcore_barrier()
    pltpu.sync_copy(acc.at[pl.ds(r0, _RPT)], outp_hbm.at[c, pl.ds(r0, _RPT)])
    pltpu.sync_copy(den, outd_hbm.at[wid])


def _sc_scatter(msg, exv, dst, zeros2, zeros1):
    mesh = plsc.VectorSubcoreMesh(
        core_axis_name="c", subcore_axis_name="s",
        num_cores=_NC, num_subcores=_NS)
    fn = pl.kernel(
        _sc_scatter_body,
        out_type=[
            jax.ShapeDtypeStruct((_NC, _NPAD, D), _f32),
            jax.ShapeDtypeStruct((_NW, _NPAD), _f32),
        ],
        mesh=mesh,
        compiler_params=pltpu.CompilerParams(needs_layout_passes=False),
        scratch_types=[
            pltpu.VMEM((_CH,), jnp.int32),
            pltpu.VMEM((_CH, D), _f32),
            pltpu.VMEM((_CH,), _f32),
            pltpu.VMEM((_NPAD,), _f32),
            pltpu.VMEM_SHARED((_NPAD, D), _f32),
            pltpu.SemaphoreType.DMA,
        ],
    )
    return fn(msg, exv, dst, zeros2, zeros1)


def _head_body(p0_ref, p1_ref, dpart_ref, wvo_ref, bvo_ref,
               cw1_ref, cb1_ref, lng_ref, lnb_ref, cw2_ref, cb2_ref,
               tw1_ref, tb1_ref, tw2_ref, tb2_ref,
               sw1_ref, sb1_ref, sw2_ref, sb2_ref,
               logits_ref, traj_ref, diff_ref, rep_ref):
    p0 = p0_ref[0]
    p1 = p1_ref[0]
    num = p0 + p1
    ones = jnp.ones((_NW, 1), _f32)
    den = lax.dot_general(dpart_ref[...], ones, (((0,), (0,)), ((), ())),
                          preferred_element_type=_f32)
    agg = num / (den + 1e-16)
    rep = jnp.dot(agg, wvo_ref[...], preferred_element_type=_f32) + bvo_ref[...]
    rep_ref[...] = rep
    u = jnp.dot(rep, cw1_ref[...], preferred_element_type=_f32) + cb1_ref[...]
    mu = jnp.mean(u, axis=-1, keepdims=True)
    var = jnp.mean((u - mu) * (u - mu), axis=-1, keepdims=True)
    un = (u - mu) * lax.rsqrt(var + 1e-5) * lng_ref[...] + lnb_ref[...]
    cvec = _gelu(un)
    logits_ref[...] = jnp.dot(cvec, cw2_ref[...], preferred_element_type=_f32) + cb2_ref[...]
    t2 = _gelu(jnp.dot(rep, tw1_ref[...], preferred_element_type=_f32) + tb1_ref[...])
    traj_ref[...] = jnp.dot(t2, tw2_ref[...], preferred_element_type=_f32) + tb2_ref[...]
    s2 = _gelu(jnp.dot(rep, sw1_ref[...], preferred_element_type=_f32) + sb1_ref[...])
    diff_ref[...] = jax.nn.sigmoid(
        jnp.dot(s2, sw2_ref[...], preferred_element_type=_f32) + sb2_ref[...])


def _head_stage(parts, dparts, wvo, bvo, cw1, cb1, lng, lnb, cw2, cb2,
                tw1, tb1, tw2, tb2, sw1, sb1, sw2, sb2):
    bn = 1024
    hd2 = D // 2
    full = lambda r, c: pl.BlockSpec((r, c), lambda i: (0, 0))
    return pl.pallas_call(
        _head_body,
        grid=(_NPAD // bn,),
        in_specs=[
            pl.BlockSpec((1, bn, D), lambda i: (0, i, 0)),
            pl.BlockSpec((1, bn, D), lambda i: (1, i, 0)),
            pl.BlockSpec((_NW, bn), lambda i: (0, i)),
            full(D, D), full(1, D),
            full(D, D), full(1, D), full(1, D), full(1, D), full(D, NF), full(1, NF),
            full(D, D), full(1, D), full(D, FE), full(1, FE),
            full(D, hd2), full(1, hd2), full(hd2, 1), full(1, 1),
        ],
        out_specs=[
            pl.BlockSpec((bn, NF), lambda i: (i, 0)),
            pl.BlockSpec((bn, FE), lambda i: (i, 0)),
            pl.BlockSpec((bn, 1), lambda i: (i, 0)),
            pl.BlockSpec((bn, D), lambda i: (i, 0)),
        ],
        out_shape=[
            jax.ShapeDtypeStruct((_NPAD, NF), _f32),
            jax.ShapeDtypeStruct((_NPAD, FE), _f32),
            jax.ShapeDtypeStruct((_NPAD, 1), _f32),
            jax.ShapeDtypeStruct((_NPAD, D), _f32),
        ],
    )(parts, parts, dparts, wvo, bvo, cw1, cb1, lng, lnb, cw2, cb2,
      tw1, tb1, tw2, tb2, sw1, sb1, sw2, sb2)


def kernel(z, edge_index, lr_scores, params):
    p = params
    src = edge_index[0]
    dst = edge_index[1]
    w1a = p['lr_W1'][:D]
    w1b = p['lr_W1'][D:2 * D]
    w1c = p['lr_W1'][2 * D:2 * D + 1]
    am_w1a = p['am_W1'][:D]
    am_w1b = p['am_W1'][D:]
    wbc = jnp.concatenate([w1b, am_w1a], axis=1)
    bbc = jnp.concatenate([jnp.zeros((D,), _f32), p['am_b1']])[None, :]
    wvo = p['Wv'] @ p['Wo']
    bvo = (p['bv'] @ p['Wo'] + p['bo'])[None, :]

    a_tab, bc_tab = _make_tables(z, w1a, p['lr_b1'][None, :], wbc, bbc)
    ga, gbc = _sc_gather(a_tab, bc_tab, src, dst)
    msg, exc = _edge_stage(
        ga, gbc, lr_scores[:, None], w1c, p['lr_W2'], p['lr_b2'][None, :],
        am_w1b, p['am_W2'][:, 0][None, :], p['am_b2'][None, :])
    parts, dparts = _sc_scatter(
        msg, exc.reshape(E), dst,
        jnp.zeros((_NPAD, D), _f32), jnp.zeros((_NPAD,), _f32))
    logits, traj, diff, rep = _head_stage(
        parts, dparts, wvo, bvo,
        p['c_W1'], p['c_b1'][None, :], p['ln_g'][None, :], p['ln_b'][None, :],
        p['c_W2'], p['c_b2'][None, :],
        p['t_W1'], p['t_b1'][None, :], p['t_W2'], p['t_b2'][None, :],
        p['s_W1'], p['s_b1'][None, :], p['s_W2'], p['s_b2'][None, :])
    return (logits[:N], traj[:N], diff[:N], rep[:N])
